# Initial kernel scaffold; baseline (speedup 1.0000x reference)
import jax
import jax.numpy as jnp
from jax.experimental import pallas as pl


def kernel(X, angles, edge_index, params):
    raise NotImplementedError("write your pallas kernel here")


# speedup vs baseline: 2.8938x; 2.8938x over previous
import functools
import math

import jax
import jax.numpy as jnp
from jax.experimental import pallas as pl

_B = 128
_N = 33
_NP = 40
_H = 64
_EPS = 1e-5


def _prep_body(ei_ref, a_ref):
    ei = ei_ref[...]
    row = ei[:, 0:1]
    col = ei[:, 1:2]
    n_iota = jax.lax.broadcasted_iota(jnp.int32, (70, _NP), 1)
    R = (row == n_iota).astype(jnp.float32)
    Cm = (col == n_iota).astype(jnp.float32)
    w = jnp.where(row == col, 0.0, 1.0)
    deg = jnp.sum(R * w, axis=0, keepdims=True)
    dis = jnp.where(deg > 0, jax.lax.rsqrt(jnp.maximum(deg, 1e-12)), 0.0)
    dr = jnp.sum(R * dis, axis=1, keepdims=True)
    dc = jnp.sum(Cm * dis, axis=1, keepdims=True)
    lapw = -dr * w * dc
    a_ref[...] = jax.lax.dot_general(
        Cm, lapw * R, (((0,), (0,)), ((), ())))


def _norm_relu(x, tin, cin, s1p, s2p, g, b, count):
    s1 = jnp.sum(s1p, axis=0)
    s2 = jnp.sum(s2p, axis=0)
    mean = s1 * (1.0 / count)
    var = s2 * (1.0 / count) - mean * mean
    inv = jax.lax.rsqrt(var + _EPS)
    a = g * inv
    c = b - mean * a
    x3 = x.reshape(tin, _NP, cin)
    x3 = jnp.maximum(x3 * a.reshape(1, _NP, 1) + c.reshape(1, _NP, 1), 0.0)
    return x3.reshape(tin * _NP, cin)


def _gated_tconv(x, tin, cin, w_ref, b_ref):
    t1 = tin - 2
    xcat = jnp.concatenate(
        [x[0:t1 * _NP], x[_NP:(t1 + 1) * _NP], x[2 * _NP:(t1 + 2) * _NP]],
        axis=1)
    gts = jnp.dot(xcat, w_ref[...]) + b_ref[...]
    return jnp.maximum(
        gts[:, 0:_H] * jax.nn.sigmoid(gts[:, _H:2 * _H]) + gts[:, 2 * _H:3 * _H],
        0.0)


def _layer_body(tin, cin, count, first, *refs):
    if first:
        (x_ref, a_ref, wt1_ref, bt1_ref, wch_ref, bch_ref,
         wt2_ref, bt2_ref, ho_ref, s1_ref, s2_ref) = refs
    else:
        (x_ref, s1p_ref, s2p_ref, g_ref, b_ref, a_ref, wt1_ref, bt1_ref,
         wch_ref, bch_ref, wt2_ref, bt2_ref, ho_ref, s1_ref, s2_ref) = refs

    x = x_ref[0]
    if not first:
        x = _norm_relu(x, tin, cin, s1p_ref[...], s2p_ref[...],
                       g_ref[...], b_ref[...], count)

    t1 = tin - 2
    u = _gated_tconv(x, tin, cin, wt1_ref, bt1_ref)

    A = a_ref[...]
    u3 = u.reshape(t1, _NP, _H)
    Ab = jnp.broadcast_to(A, (t1, _NP, _NP))
    z1 = jax.lax.dot_general(Ab, u3, (((2,), (1,)), ((0,), (0,))))
    z2 = jax.lax.dot_general(Ab, z1, (((2,), (1,)), ((0,), (0,))))
    ccat = jnp.concatenate(
        [u, z1.reshape(t1 * _NP, _H), z2.reshape(t1 * _NP, _H)], axis=1)
    y = jnp.maximum(jnp.dot(ccat, wch_ref[...]) + bch_ref[...], 0.0)

    t2 = t1 - 2
    h = _gated_tconv(y, t1, _H, wt2_ref, bt2_ref)

    ho_ref[...] = h[None]
    h3 = h.reshape(t2, _NP, _H)
    hs = jnp.sum(h3, axis=0)
    h2s = jnp.sum(h3 * h3, axis=0)
    s1_ref[...] = jnp.sum(hs, axis=1, keepdims=True)[None]
    s2_ref[...] = jnp.sum(h2s, axis=1, keepdims=True)[None]


def _attn_body(tin, count, x_ref, s1p_ref, s2p_ref, g_ref, b_ref,
               win_ref, bin_ref, wout_ref, bout_ref, out_ref):
    x = x_ref[0]
    x = _norm_relu(x, tin, _H, s1p_ref[...], s2p_ref[...],
                   g_ref[...], b_ref[...], count)
    qkv = jnp.dot(x, win_ref[...]) + bin_ref[...]
    q3 = qkv[:, 0:_H].reshape(tin, _NP, _H)
    k3 = qkv[:, _H:2 * _H].reshape(tin, _NP, _H)
    v3 = qkv[:, 2 * _H:3 * _H].reshape(tin, _NP, _H)
    qn = jnp.transpose(q3, (1, 0, 2))
    kn = jnp.transpose(k3, (1, 0, 2))
    vn = jnp.transpose(v3, (1, 0, 2))

    lane = jax.lax.broadcasted_iota(jnp.int32, (1, 1, _H), 2)
    masks = [((lane >= h * 16) & (lane < h * 16 + 16)).astype(jnp.float32)
             for h in range(4)]
    qm = jnp.concatenate([qn * m for m in masks], axis=1) * 0.25

    s = jax.lax.dot_general(qm, kn, (((2,), (2,)), ((0,), (0,))))
    s = s - jnp.max(s, axis=2, keepdims=True)
    e = jnp.exp(s)
    attn = e / jnp.sum(e, axis=2, keepdims=True)
    o4 = jax.lax.dot_general(attn, vn, (((2,), (1,)), ((0,), (0,))))
    o = (o4[:, 0:tin] * masks[0] + o4[:, tin:2 * tin] * masks[1]
         + o4[:, 2 * tin:3 * tin] * masks[2] + o4[:, 3 * tin:4 * tin] * masks[3])

    nid = jax.lax.broadcasted_iota(jnp.int32, (_NP, 1, 1), 0)
    om = jnp.where(nid < _N, o, 0.0)
    pv = jnp.sum(jnp.sum(om, axis=0), axis=0, keepdims=True) * (1.0 / (_N * tin))
    out_ref[...] = (jnp.dot(pv, wout_ref[...]) + bout_ref[...])[None]


def _heads_body(pooled_ref, ang_ref, wang_ref, bang_ref, wcls_ref, bcls_ref,
                wkey_ref, bkey_ref, wreg_ref, breg_ref,
                logits_ref, key_ref, angs_ref):
    xf = (pooled_ref[...] + jnp.dot(ang_ref[...], wang_ref[...])
          + bang_ref[...])
    logits_ref[...] = jnp.dot(xf, wcls_ref[...]) + bcls_ref[...]
    key_ref[...] = jax.nn.sigmoid(jnp.dot(xf, wkey_ref[...]) + bkey_ref[...])
    angs_ref[...] = jnp.dot(xf, wreg_ref[...]) + breg_ref[...]


def _full(shape):
    nd = len(shape)
    return pl.BlockSpec(shape, lambda b, _n=nd: (0,) * _n)


def _batched(shape):
    nd = len(shape)
    return pl.BlockSpec((1,) + shape[1:], lambda b, _n=nd: (b,) + (0,) * (_n - 1))


def _tconv_w(p, w1, b1, w2, b2, w3, b3):
    def gate(w):
        return jnp.concatenate([w[:, :, 0, k].T for k in range(3)], axis=0)
    wcat = jnp.concatenate([gate(p[w1]), gate(p[w2]), gate(p[w3])], axis=1)
    bcat = jnp.concatenate([p[b1], p[b2], p[b3]])[None, :]
    return wcat, bcat


def kernel(X, angles, edge_index, params):
    f32 = jnp.float32

    ei_t = jnp.transpose(edge_index, (1, 0)).astype(jnp.int32)
    A = pl.pallas_call(
        _prep_body,
        out_shape=jax.ShapeDtypeStruct((_NP, _NP), f32),
    )(ei_t)

    layer_w = []
    for lp in params['layers']:
        wt1, bt1 = _tconv_w(lp['t1'], 'w1', 'b1', 'w2', 'b2', 'w3', 'b3')
        wt2, bt2 = _tconv_w(lp['t2'], 'w1', 'b1', 'w2', 'b2', 'w3', 'b3')
        w0, w1, w2 = lp['cheb_w']
        wch = jnp.concatenate([w0 - w2, w1, 2.0 * w2], axis=0)
        bch = lp['cheb_b'][None, :]
        g = jnp.pad(lp['bn_g'], (0, _NP - _N))[:, None]
        b = jnp.pad(lp['bn_b'], (0, _NP - _N))[:, None]
        layer_w.append((wt1, bt1, wch, bch, wt2, bt2, g, b))

    Xp = jnp.pad(X, ((0, 0), (0, 0), (0, _NP - _N), (0, 0)))
    Xr = Xp.reshape(_B, 64 * _NP, 2)

    tins = [64, 60, 56]
    h = Xr
    stats = None
    for li in range(3):
        tin = tins[li]
        t2 = tin - 4
        cin = 2 if li == 0 else _H
        wt1, bt1, wch, bch, wt2, bt2, g, b = layer_w[li]
        first = li == 0
        count = _B * tin * _H
        body = functools.partial(_layer_body, tin, cin, count, first)
        in_specs = [_batched((_B, tin * _NP, cin))]
        ins = [h]
        if not first:
            in_specs += [_full((_B, _NP, 1))] * 2 + [_full((_NP, 1))] * 2
            ins += [stats[0], stats[1], g, b]
        in_specs += [_full((_NP, _NP)),
                     _full(wt1.shape), _full(bt1.shape),
                     _full(wch.shape), _full(bch.shape),
                     _full(wt2.shape), _full(bt2.shape)]
        ins += [A, wt1, bt1, wch, bch, wt2, bt2]
        h, s1, s2 = pl.pallas_call(
            body,
            grid=(_B,),
            in_specs=in_specs,
            out_specs=[_batched((_B, t2 * _NP, _H)),
                       _batched((_B, _NP, 1)), _batched((_B, _NP, 1))],
            out_shape=[jax.ShapeDtypeStruct((_B, t2 * _NP, _H), f32),
                       jax.ShapeDtypeStruct((_B, _NP, 1), f32),
                       jax.ShapeDtypeStruct((_B, _NP, 1), f32)],
        )(*ins)
        stats = (s1, s2)

    ap = params['attn']
    g3 = layer_w[2][6]
    b3 = layer_w[2][7]
    win = ap['in_w'].T
    bin_ = ap['in_b'][None, :]
    wout = ap['out_w'].T
    bout = ap['out_b'][None, :]
    tin = 52
    body = functools.partial(_attn_body, tin, _B * tin * _H)
    pooled = pl.pallas_call(
        body,
        grid=(_B,),
        in_specs=[_batched((_B, tin * _NP, _H)),
                  _full((_B, _NP, 1)), _full((_B, _NP, 1)),
                  _full((_NP, 1)), _full((_NP, 1)),
                  _full(win.shape), _full(bin_.shape),
                  _full(wout.shape), _full(bout.shape)],
        out_specs=_batched((_B, 1, _H)),
        out_shape=jax.ShapeDtypeStruct((_B, 1, _H), f32),
    )(h, stats[0], stats[1], g3, b3, win, bin_, wout, bout)
    pooled = pooled.reshape(_B, _H)

    hp = params['heads']
    wang = hp['angle_w'].T
    bang = hp['angle_b'][None, :]
    wcls = hp['cls_w'].T
    bcls = hp['cls_b'][None, :]
    wkey = hp['key_w'].T
    bkey = hp['key_b'][None, :]
    wreg = hp['reg_w'].T
    breg = hp['reg_b'][None, :]
    logits, key_action, angle_scores = pl.pallas_call(
        _heads_body,
        out_shape=[jax.ShapeDtypeStruct((_B, 15), f32),
                   jax.ShapeDtypeStruct((_B, 1), f32),
                   jax.ShapeDtypeStruct((_B, 4), f32)],
    )(pooled, angles, wang, bang, wcls, bcls, wkey, bkey, wreg, breg)
    return (logits, key_action, angle_scores)


# speedup vs baseline: 3.2400x; 1.1196x over previous
import functools
import math

import jax
import jax.numpy as jnp
from jax.experimental import pallas as pl

_B = 128
_NB = 4
_N = 33
_NP = 40
_H = 64
_EPS = 1e-5


def _prep_body(ei_ref, a_ref):
    ei = ei_ref[...]
    row = ei[:, 0:1]
    col = ei[:, 1:2]
    n_iota = jax.lax.broadcasted_iota(jnp.int32, (70, _NP), 1)
    R = (row == n_iota).astype(jnp.float32)
    Cm = (col == n_iota).astype(jnp.float32)
    w = jnp.where(row == col, 0.0, 1.0)
    deg = jnp.sum(R * w, axis=0, keepdims=True)
    dis = jnp.where(deg > 0, jax.lax.rsqrt(jnp.maximum(deg, 1e-12)), 0.0)
    dr = jnp.sum(R * dis, axis=1, keepdims=True)
    dc = jnp.sum(Cm * dis, axis=1, keepdims=True)
    lapw = -dr * w * dc
    a_ref[...] = jax.lax.dot_general(
        Cm, lapw * R, (((0,), (0,)), ((), ())))


def _gated_tconv(x, tin, cin, w_ref, b_ref):
    t1 = tin - 2
    xcat = jnp.concatenate(
        [x[0:t1 * _NP], x[_NP:(t1 + 1) * _NP], x[2 * _NP:(t1 + 2) * _NP]],
        axis=1)
    gts = jnp.dot(xcat, w_ref[...]) + b_ref[...]
    return jnp.maximum(
        gts[:, 0:_H] * jax.nn.sigmoid(gts[:, _H:2 * _H]) + gts[:, 2 * _H:3 * _H],
        0.0)


def _layer_body(tin, cin, count, first, *refs):
    if first:
        (x_ref, a_ref, wt1_ref, bt1_ref, wch_ref, bch_ref,
         wt2_ref, bt2_ref, ho_ref, s1_ref, s2_ref) = refs
    else:
        (x_ref, s1p_ref, s2p_ref, g_ref, b_ref, a_ref, wt1_ref, bt1_ref,
         wch_ref, bch_ref, wt2_ref, bt2_ref, ho_ref, s1_ref, s2_ref) = refs

    t1 = tin - 2
    t2 = t1 - 2
    A = a_ref[...]
    if not first:
        s1 = jnp.sum(s1p_ref[...], axis=0)
        s2 = jnp.sum(s2p_ref[...], axis=0)
        mean = s1 * (1.0 / count)
        var = s2 * (1.0 / count) - mean * mean
        inv = jax.lax.rsqrt(var + _EPS)
        aff_a = (g_ref[...] * inv).reshape(1, _NP, 1)
        aff_c = (b_ref[...] - mean * g_ref[...] * inv).reshape(1, _NP, 1)

    for i in range(_NB):
        x = x_ref[i]
        if not first:
            x3 = x.reshape(tin, _NP, cin)
            x = jnp.maximum(x3 * aff_a + aff_c, 0.0).reshape(tin * _NP, cin)

        u = _gated_tconv(x, tin, cin, wt1_ref, bt1_ref)

        u3 = u.reshape(t1, _NP, _H)
        Ab = jnp.broadcast_to(A, (t1, _NP, _NP))
        z1 = jax.lax.dot_general(Ab, u3, (((2,), (1,)), ((0,), (0,))))
        z2 = jax.lax.dot_general(Ab, z1, (((2,), (1,)), ((0,), (0,))))
        ccat = jnp.concatenate(
            [u, z1.reshape(t1 * _NP, _H), z2.reshape(t1 * _NP, _H)], axis=1)
        y = jnp.maximum(jnp.dot(ccat, wch_ref[...]) + bch_ref[...], 0.0)

        h = _gated_tconv(y, t1, _H, wt2_ref, bt2_ref)

        ho_ref[i] = h
        h3 = h.reshape(t2, _NP, _H)
        hs = jnp.sum(h3, axis=0)
        h2s = jnp.sum(h3 * h3, axis=0)
        s1_ref[i] = jnp.sum(hs, axis=1, keepdims=True)
        s2_ref[i] = jnp.sum(h2s, axis=1, keepdims=True)


def _attn_body(tin, count, x_ref, s1p_ref, s2p_ref, g_ref, b_ref,
               win_ref, bin_ref, wout_ref, bout_ref, out_ref):
    s1 = jnp.sum(s1p_ref[...], axis=0)
    s2 = jnp.sum(s2p_ref[...], axis=0)
    mean = s1 * (1.0 / count)
    var = s2 * (1.0 / count) - mean * mean
    inv = jax.lax.rsqrt(var + _EPS)
    aff_a = (g_ref[...] * inv).reshape(1, _NP, 1)
    aff_c = (b_ref[...] - mean * g_ref[...] * inv).reshape(1, _NP, 1)

    lane = jax.lax.broadcasted_iota(jnp.int32, (1, 1, _H), 2)
    masks = [((lane >= h * 16) & (lane < h * 16 + 16)).astype(jnp.float32)
             for h in range(4)]
    nid = jax.lax.broadcasted_iota(jnp.int32, (_NP, 1, 1), 0)

    for i in range(_NB):
        x3 = x_ref[i].reshape(tin, _NP, _H)
        x = jnp.maximum(x3 * aff_a + aff_c, 0.0).reshape(tin * _NP, _H)
        qkv = jnp.dot(x, win_ref[...]) + bin_ref[...]
        q3 = qkv[:, 0:_H].reshape(tin, _NP, _H)
        k3 = qkv[:, _H:2 * _H].reshape(tin, _NP, _H)
        v3 = qkv[:, 2 * _H:3 * _H].reshape(tin, _NP, _H)
        qn = jnp.transpose(q3, (1, 0, 2))
        kn = jnp.transpose(k3, (1, 0, 2))
        vn = jnp.transpose(v3, (1, 0, 2))

        qm = jnp.concatenate([qn * m for m in masks], axis=1) * 0.25

        s = jax.lax.dot_general(qm, kn, (((2,), (2,)), ((0,), (0,))))
        s = s - jnp.max(s, axis=2, keepdims=True)
        e = jnp.exp(s)
        attn = e / jnp.sum(e, axis=2, keepdims=True)
        o4 = jax.lax.dot_general(attn, vn, (((2,), (1,)), ((0,), (0,))))
        o = (o4[:, 0:tin] * masks[0] + o4[:, tin:2 * tin] * masks[1]
             + o4[:, 2 * tin:3 * tin] * masks[2]
             + o4[:, 3 * tin:4 * tin] * masks[3])

        om = jnp.where(nid < _N, o, 0.0)
        pv = (jnp.sum(jnp.sum(om, axis=0), axis=0, keepdims=True)
              * (1.0 / (_N * tin)))
        out_ref[i] = jnp.dot(pv, wout_ref[...]) + bout_ref[...]


def _heads_body(pooled_ref, ang_ref, wang_ref, bang_ref, wcls_ref, bcls_ref,
                wkey_ref, bkey_ref, wreg_ref, breg_ref,
                logits_ref, key_ref, angs_ref):
    xf = (pooled_ref[...] + jnp.dot(ang_ref[...], wang_ref[...])
          + bang_ref[...])
    logits_ref[...] = jnp.dot(xf, wcls_ref[...]) + bcls_ref[...]
    key_ref[...] = jax.nn.sigmoid(jnp.dot(xf, wkey_ref[...]) + bkey_ref[...])
    angs_ref[...] = jnp.dot(xf, wreg_ref[...]) + breg_ref[...]


def _full(shape):
    nd = len(shape)
    return pl.BlockSpec(shape, lambda b, _n=nd: (0,) * _n)


def _batched(shape):
    nd = len(shape)
    return pl.BlockSpec((_NB,) + shape[1:],
                        lambda b, _n=nd: (b,) + (0,) * (_n - 1))


def _tconv_w(p, w1, b1, w2, b2, w3, b3):
    def gate(w):
        return jnp.concatenate([w[:, :, 0, k].T for k in range(3)], axis=0)
    wcat = jnp.concatenate([gate(p[w1]), gate(p[w2]), gate(p[w3])], axis=1)
    bcat = jnp.concatenate([p[b1], p[b2], p[b3]])[None, :]
    return wcat, bcat


def kernel(X, angles, edge_index, params):
    f32 = jnp.float32

    ei_t = jnp.transpose(edge_index, (1, 0)).astype(jnp.int32)
    A = pl.pallas_call(
        _prep_body,
        out_shape=jax.ShapeDtypeStruct((_NP, _NP), f32),
    )(ei_t)

    layer_w = []
    for lp in params['layers']:
        wt1, bt1 = _tconv_w(lp['t1'], 'w1', 'b1', 'w2', 'b2', 'w3', 'b3')
        wt2, bt2 = _tconv_w(lp['t2'], 'w1', 'b1', 'w2', 'b2', 'w3', 'b3')
        w0, w1, w2 = lp['cheb_w']
        wch = jnp.concatenate([w0 - w2, w1, 2.0 * w2], axis=0)
        bch = lp['cheb_b'][None, :]
        g = jnp.pad(lp['bn_g'], (0, _NP - _N))[:, None]
        b = jnp.pad(lp['bn_b'], (0, _NP - _N))[:, None]
        layer_w.append((wt1, bt1, wch, bch, wt2, bt2, g, b))

    Xp = jnp.pad(X, ((0, 0), (0, 0), (0, _NP - _N), (0, 0)))
    Xr = Xp.reshape(_B, 64 * _NP, 2)

    tins = [64, 60, 56]
    h = Xr
    stats = None
    for li in range(3):
        tin = tins[li]
        t2 = tin - 4
        cin = 2 if li == 0 else _H
        wt1, bt1, wch, bch, wt2, bt2, g, b = layer_w[li]
        first = li == 0
        count = _B * tin * _H
        body = functools.partial(_layer_body, tin, cin, count, first)
        in_specs = [_batched((_B, tin * _NP, cin))]
        ins = [h]
        if not first:
            in_specs += [_full((_B, _NP, 1))] * 2 + [_full((_NP, 1))] * 2
            ins += [stats[0], stats[1], g, b]
        in_specs += [_full((_NP, _NP)),
                     _full(wt1.shape), _full(bt1.shape),
                     _full(wch.shape), _full(bch.shape),
                     _full(wt2.shape), _full(bt2.shape)]
        ins += [A, wt1, bt1, wch, bch, wt2, bt2]
        h, s1, s2 = pl.pallas_call(
            body,
            grid=(_B // _NB,),
            in_specs=in_specs,
            out_specs=[_batched((_B, t2 * _NP, _H)),
                       _batched((_B, _NP, 1)), _batched((_B, _NP, 1))],
            out_shape=[jax.ShapeDtypeStruct((_B, t2 * _NP, _H), f32),
                       jax.ShapeDtypeStruct((_B, _NP, 1), f32),
                       jax.ShapeDtypeStruct((_B, _NP, 1), f32)],
        )(*ins)
        stats = (s1, s2)

    ap = params['attn']
    g3 = layer_w[2][6]
    b3 = layer_w[2][7]
    win = ap['in_w'].T
    bin_ = ap['in_b'][None, :]
    wout = ap['out_w'].T
    bout = ap['out_b'][None, :]
    tin = 52
    body = functools.partial(_attn_body, tin, _B * tin * _H)
    pooled = pl.pallas_call(
        body,
        grid=(_B // _NB,),
        in_specs=[_batched((_B, tin * _NP, _H)),
                  _full((_B, _NP, 1)), _full((_B, _NP, 1)),
                  _full((_NP, 1)), _full((_NP, 1)),
                  _full(win.shape), _full(bin_.shape),
                  _full(wout.shape), _full(bout.shape)],
        out_specs=_batched((_B, 1, _H)),
        out_shape=jax.ShapeDtypeStruct((_B, 1, _H), f32),
    )(h, stats[0], stats[1], g3, b3, win, bin_, wout, bout)
    pooled = pooled.reshape(_B, _H)

    hp = params['heads']
    wang = hp['angle_w'].T
    bang = hp['angle_b'][None, :]
    wcls = hp['cls_w'].T
    bcls = hp['cls_b'][None, :]
    wkey = hp['key_w'].T
    bkey = hp['key_b'][None, :]
    wreg = hp['reg_w'].T
    breg = hp['reg_b'][None, :]
    logits, key_action, angle_scores = pl.pallas_call(
        _heads_body,
        out_shape=[jax.ShapeDtypeStruct((_B, 15), f32),
                   jax.ShapeDtypeStruct((_B, 1), f32),
                   jax.ShapeDtypeStruct((_B, 4), f32)],
    )(pooled, angles, wang, bang, wcls, bcls, wkey, bkey, wreg, breg)
    return (logits, key_action, angle_scores)


# speedup vs baseline: 3.3065x; 1.0205x over previous
import functools
import math

import jax
import jax.numpy as jnp
from jax.experimental import pallas as pl

_B = 128
_NB = 4
_N = 33
_NP = 40
_H = 64
_EPS = 1e-5


def _prep_body(ei_ref, a_ref):
    ei = ei_ref[...]
    row = ei[:, 0:1]
    col = ei[:, 1:2]
    n_iota = jax.lax.broadcasted_iota(jnp.int32, (70, _NP), 1)
    R = (row == n_iota).astype(jnp.float32)
    Cm = (col == n_iota).astype(jnp.float32)
    w = jnp.where(row == col, 0.0, 1.0)
    deg = jnp.sum(R * w, axis=0, keepdims=True)
    dis = jnp.where(deg > 0, jax.lax.rsqrt(jnp.maximum(deg, 1e-12)), 0.0)
    dr = jnp.sum(R * dis, axis=1, keepdims=True)
    dc = jnp.sum(Cm * dis, axis=1, keepdims=True)
    lapw = -dr * w * dc
    a_ref[...] = jax.lax.dot_general(
        Cm, lapw * R, (((0,), (0,)), ((), ())))


def _sigmoid(x):
    return 1.0 / (1.0 + jnp.exp(-x))


def _gated_tconv(x, tin, cin, w_ref, b_ref):
    t1 = tin - 2
    xcat = jnp.concatenate(
        [x[0:t1 * _NP], x[_NP:(t1 + 1) * _NP], x[2 * _NP:(t1 + 2) * _NP]],
        axis=1)
    gts = jnp.dot(xcat, w_ref[...]) + b_ref[...]
    return jnp.maximum(
        gts[:, 0:_H] * _sigmoid(gts[:, _H:2 * _H]) + gts[:, 2 * _H:3 * _H],
        0.0)


def _stats_body(count, s1p_ref, s2p_ref, g_ref, b_ref, a_ref, c_ref):
    s1 = jnp.sum(s1p_ref[...], axis=0)
    s2 = jnp.sum(s2p_ref[...], axis=0)
    mean = s1 * (1.0 / count)
    var = s2 * (1.0 / count) - mean * mean
    inv = jax.lax.rsqrt(var + _EPS)
    a_ref[...] = g_ref[...] * inv
    c_ref[...] = b_ref[...] - mean * g_ref[...] * inv


def _bn_affine(count, s1, s2, g, b):
    return pl.pallas_call(
        functools.partial(_stats_body, count),
        out_shape=[jax.ShapeDtypeStruct((_NP, 1), jnp.float32)] * 2,
    )(s1, s2, g, b)


def _layer_body(tin, cin, count, first, *refs):
    if first:
        (x_ref, a_ref, wt1_ref, bt1_ref, wch_ref, bch_ref,
         wt2_ref, bt2_ref, ho_ref, s1_ref, s2_ref) = refs
    else:
        (x_ref, affa_ref, affc_ref, a_ref, wt1_ref, bt1_ref,
         wch_ref, bch_ref, wt2_ref, bt2_ref, ho_ref, s1_ref, s2_ref) = refs

    t1 = tin - 2
    t2 = t1 - 2
    A = a_ref[...]
    if not first:
        aff_a = affa_ref[...].reshape(1, _NP, 1)
        aff_c = affc_ref[...].reshape(1, _NP, 1)

    for i in range(_NB):
        x = x_ref[i]
        if not first:
            x3 = x.reshape(tin, _NP, cin)
            x = jnp.maximum(x3 * aff_a + aff_c, 0.0).reshape(tin * _NP, cin)

        u = _gated_tconv(x, tin, cin, wt1_ref, bt1_ref)

        u3 = u.reshape(t1, _NP, _H)
        Ab = jnp.broadcast_to(A, (t1, _NP, _NP))
        z1 = jax.lax.dot_general(Ab, u3, (((2,), (1,)), ((0,), (0,))))
        z2 = jax.lax.dot_general(Ab, z1, (((2,), (1,)), ((0,), (0,))))
        ccat = jnp.concatenate(
            [u, z1.reshape(t1 * _NP, _H), z2.reshape(t1 * _NP, _H)], axis=1)
        y = jnp.maximum(jnp.dot(ccat, wch_ref[...]) + bch_ref[...], 0.0)

        h = _gated_tconv(y, t1, _H, wt2_ref, bt2_ref)

        ho_ref[i] = h
        h3 = h.reshape(t2, _NP, _H)
        hs = jnp.sum(h3, axis=0)
        h2s = jnp.sum(h3 * h3, axis=0)
        s1_ref[i] = jnp.sum(hs, axis=1, keepdims=True)
        s2_ref[i] = jnp.sum(h2s, axis=1, keepdims=True)


def _attn_body(tin, x_ref, affa_ref, affc_ref,
               win_ref, bin_ref, wout_ref, bout_ref, out_ref):
    aff_a = affa_ref[...].reshape(1, _NP, 1)
    aff_c = affc_ref[...].reshape(1, _NP, 1)

    lane = jax.lax.broadcasted_iota(jnp.int32, (1, 1, _H), 2)
    masks = [((lane >= h * 16) & (lane < h * 16 + 16)).astype(jnp.float32)
             for h in range(4)]
    nid = jax.lax.broadcasted_iota(jnp.int32, (_NP, 1, 1), 0)

    for i in range(_NB):
        x3 = x_ref[i].reshape(tin, _NP, _H)
        x = jnp.maximum(x3 * aff_a + aff_c, 0.0).reshape(tin * _NP, _H)
        qkv = jnp.dot(x, win_ref[...]) + bin_ref[...]
        q3 = qkv[:, 0:_H].reshape(tin, _NP, _H)
        k3 = qkv[:, _H:2 * _H].reshape(tin, _NP, _H)
        v3 = qkv[:, 2 * _H:3 * _H].reshape(tin, _NP, _H)
        qn = jnp.transpose(q3, (1, 0, 2))
        kn = jnp.transpose(k3, (1, 0, 2))
        vn = jnp.transpose(v3, (1, 0, 2))

        qm = jnp.concatenate([qn * m for m in masks], axis=1) * 0.25

        s = jax.lax.dot_general(qm, kn, (((2,), (2,)), ((0,), (0,))))
        s = s - jnp.max(s, axis=2, keepdims=True)
        e = jnp.exp(s)
        attn = e / jnp.sum(e, axis=2, keepdims=True)
        o4 = jax.lax.dot_general(attn, vn, (((2,), (1,)), ((0,), (0,))))
        o = (o4[:, 0:tin] * masks[0] + o4[:, tin:2 * tin] * masks[1]
             + o4[:, 2 * tin:3 * tin] * masks[2]
             + o4[:, 3 * tin:4 * tin] * masks[3])

        om = jnp.where(nid < _N, o, 0.0)
        pv = (jnp.sum(jnp.sum(om, axis=0), axis=0, keepdims=True)
              * (1.0 / (_N * tin)))
        out_ref[i] = jnp.dot(pv, wout_ref[...]) + bout_ref[...]


def _heads_body(pooled_ref, ang_ref, wang_ref, bang_ref, wcls_ref, bcls_ref,
                wkey_ref, bkey_ref, wreg_ref, breg_ref,
                logits_ref, key_ref, angs_ref):
    xf = (pooled_ref[...] + jnp.dot(ang_ref[...], wang_ref[...])
          + bang_ref[...])
    logits_ref[...] = jnp.dot(xf, wcls_ref[...]) + bcls_ref[...]
    key_ref[...] = jax.nn.sigmoid(jnp.dot(xf, wkey_ref[...]) + bkey_ref[...])
    angs_ref[...] = jnp.dot(xf, wreg_ref[...]) + breg_ref[...]


def _full(shape):
    nd = len(shape)
    return pl.BlockSpec(shape, lambda b, _n=nd: (0,) * _n)


def _batched(shape):
    nd = len(shape)
    return pl.BlockSpec((_NB,) + shape[1:],
                        lambda b, _n=nd: (b,) + (0,) * (_n - 1))


def _tconv_w(p, w1, b1, w2, b2, w3, b3):
    def gate(w):
        return jnp.concatenate([w[:, :, 0, k].T for k in range(3)], axis=0)
    wcat = jnp.concatenate([gate(p[w1]), gate(p[w2]), gate(p[w3])], axis=1)
    bcat = jnp.concatenate([p[b1], p[b2], p[b3]])[None, :]
    return wcat, bcat


def kernel(X, angles, edge_index, params):
    f32 = jnp.float32

    ei_t = jnp.transpose(edge_index, (1, 0)).astype(jnp.int32)
    A = pl.pallas_call(
        _prep_body,
        out_shape=jax.ShapeDtypeStruct((_NP, _NP), f32),
    )(ei_t)

    layer_w = []
    for lp in params['layers']:
        wt1, bt1 = _tconv_w(lp['t1'], 'w1', 'b1', 'w2', 'b2', 'w3', 'b3')
        wt2, bt2 = _tconv_w(lp['t2'], 'w1', 'b1', 'w2', 'b2', 'w3', 'b3')
        w0, w1, w2 = lp['cheb_w']
        wch = jnp.concatenate([w0 - w2, w1, 2.0 * w2], axis=0)
        bch = lp['cheb_b'][None, :]
        g = jnp.pad(lp['bn_g'], (0, _NP - _N))[:, None]
        b = jnp.pad(lp['bn_b'], (0, _NP - _N))[:, None]
        layer_w.append((wt1, bt1, wch, bch, wt2, bt2, g, b))

    Xp = jnp.pad(X, ((0, 0), (0, 0), (0, _NP - _N), (0, 0)))
    Xr = Xp.reshape(_B, 64 * _NP, 2)

    tins = [64, 60, 56]
    h = Xr
    stats = None
    for li in range(3):
        tin = tins[li]
        t2 = tin - 4
        cin = 2 if li == 0 else _H
        wt1, bt1, wch, bch, wt2, bt2, g, b = layer_w[li]
        first = li == 0
        count = _B * tin * _H
        body = functools.partial(_layer_body, tin, cin, count, first)
        in_specs = [_batched((_B, tin * _NP, cin))]
        ins = [h]
        if not first:
            gp, bp = layer_w[li - 1][6], layer_w[li - 1][7]
            aff_a, aff_c = _bn_affine(count, stats[0], stats[1], gp, bp)
            in_specs += [_full((_NP, 1))] * 2
            ins += [aff_a, aff_c]
        in_specs += [_full((_NP, _NP)),
                     _full(wt1.shape), _full(bt1.shape),
                     _full(wch.shape), _full(bch.shape),
                     _full(wt2.shape), _full(bt2.shape)]
        ins += [A, wt1, bt1, wch, bch, wt2, bt2]
        h, s1, s2 = pl.pallas_call(
            body,
            grid=(_B // _NB,),
            in_specs=in_specs,
            out_specs=[_batched((_B, t2 * _NP, _H)),
                       _batched((_B, _NP, 1)), _batched((_B, _NP, 1))],
            out_shape=[jax.ShapeDtypeStruct((_B, t2 * _NP, _H), f32),
                       jax.ShapeDtypeStruct((_B, _NP, 1), f32),
                       jax.ShapeDtypeStruct((_B, _NP, 1), f32)],
        )(*ins)
        stats = (s1, s2)

    ap = params['attn']
    g3 = layer_w[2][6]
    b3 = layer_w[2][7]
    win = ap['in_w'].T
    bin_ = ap['in_b'][None, :]
    wout = ap['out_w'].T
    bout = ap['out_b'][None, :]
    tin = 52
    aff_a, aff_c = _bn_affine(_B * tin * _H, stats[0], stats[1], g3, b3)
    body = functools.partial(_attn_body, tin)
    pooled = pl.pallas_call(
        body,
        grid=(_B // _NB,),
        in_specs=[_batched((_B, tin * _NP, _H)),
                  _full((_NP, 1)), _full((_NP, 1)),
                  _full(win.shape), _full(bin_.shape),
                  _full(wout.shape), _full(bout.shape)],
        out_specs=_batched((_B, 1, _H)),
        out_shape=jax.ShapeDtypeStruct((_B, 1, _H), f32),
    )(h, aff_a, aff_c, win, bin_, wout, bout)
    pooled = pooled.reshape(_B, _H)

    hp = params['heads']
    wang = hp['angle_w'].T
    bang = hp['angle_b'][None, :]
    wcls = hp['cls_w'].T
    bcls = hp['cls_b'][None, :]
    wkey = hp['key_w'].T
    bkey = hp['key_b'][None, :]
    wreg = hp['reg_w'].T
    breg = hp['reg_b'][None, :]
    logits, key_action, angle_scores = pl.pallas_call(
        _heads_body,
        out_shape=[jax.ShapeDtypeStruct((_B, 15), f32),
                   jax.ShapeDtypeStruct((_B, 1), f32),
                   jax.ShapeDtypeStruct((_B, 4), f32)],
    )(pooled, angles, wang, bang, wcls, bcls, wkey, bkey, wreg, breg)
    return (logits, key_action, angle_scores)


# speedup vs baseline: 3.5485x; 1.0732x over previous
import functools
import math

import jax
import jax.numpy as jnp
from jax.experimental import pallas as pl

_B = 128
_NB = 4
_N = 33
_NP = 40
_H = 64
_EPS = 1e-5


def _prep_body(ei_ref, a_ref):
    ei = ei_ref[...]
    row = ei[:, 0:1]
    col = ei[:, 1:2]
    n_iota = jax.lax.broadcasted_iota(jnp.int32, (70, _NP), 1)
    R = (row == n_iota).astype(jnp.float32)
    Cm = (col == n_iota).astype(jnp.float32)
    w = jnp.where(row == col, 0.0, 1.0)
    deg = jnp.sum(R * w, axis=0, keepdims=True)
    dis = jnp.where(deg > 0, jax.lax.rsqrt(jnp.maximum(deg, 1e-12)), 0.0)
    dr = jnp.sum(R * dis, axis=1, keepdims=True)
    dc = jnp.sum(Cm * dis, axis=1, keepdims=True)
    lapw = -dr * w * dc
    a_ref[...] = jax.lax.dot_general(
        Cm, lapw * R, (((0,), (0,)), ((), ())))


def _sigmoid(x):
    return 0.5 * jnp.tanh(0.5 * x) + 0.5


def _gated_tconv(x, tin, cin, w_ref, b_ref):
    t1 = tin - 2
    xcat = jnp.concatenate(
        [x[0:t1 * _NP], x[_NP:(t1 + 1) * _NP], x[2 * _NP:(t1 + 2) * _NP]],
        axis=1)
    gts = jnp.dot(xcat, w_ref[...],
                  preferred_element_type=jnp.float32) + b_ref[...]
    return jnp.maximum(
        gts[:, 0:_H] * _sigmoid(gts[:, _H:2 * _H]) + gts[:, 2 * _H:3 * _H],
        0.0)


def _stats_body(count, s1p_ref, s2p_ref, g_ref, b_ref, a_ref, c_ref):
    s1 = jnp.sum(s1p_ref[...], axis=0)
    s2 = jnp.sum(s2p_ref[...], axis=0)
    mean = s1 * (1.0 / count)
    var = s2 * (1.0 / count) - mean * mean
    inv = jax.lax.rsqrt(var + _EPS)
    a_ref[...] = g_ref[...] * inv
    c_ref[...] = b_ref[...] - mean * g_ref[...] * inv


def _bn_affine(count, s1, s2, g, b):
    return pl.pallas_call(
        functools.partial(_stats_body, count),
        out_shape=[jax.ShapeDtypeStruct((_NP, 1), jnp.float32)] * 2,
    )(s1, s2, g, b)


def _layer_body(tin, cin, count, first, *refs):
    if first:
        (x_ref, a_ref, wt1_ref, bt1_ref, wch_ref, bch_ref,
         wt2_ref, bt2_ref, ho_ref, s1_ref, s2_ref) = refs
    else:
        (x_ref, affa_ref, affc_ref, a_ref, wt1_ref, bt1_ref,
         wch_ref, bch_ref, wt2_ref, bt2_ref, ho_ref, s1_ref, s2_ref) = refs

    t1 = tin - 2
    t2 = t1 - 2
    A = a_ref[...]
    if not first:
        aff_a = affa_ref[...].reshape(1, _NP, 1)
        aff_c = affc_ref[...].reshape(1, _NP, 1)

    for i in range(_NB):
        x = x_ref[i]
        if not first:
            x3 = x.reshape(tin, _NP, cin)
            x = jnp.maximum(x3 * aff_a + aff_c, 0.0).reshape(tin * _NP, cin)

        u = _gated_tconv(x, tin, cin, wt1_ref, bt1_ref)

        u3 = u.reshape(t1, _NP, _H)
        Ab = jnp.broadcast_to(A, (t1, _NP, _NP))
        z1 = jax.lax.dot_general(Ab, u3, (((2,), (1,)), ((0,), (0,))),
                                 preferred_element_type=jnp.float32)
        z2 = jax.lax.dot_general(Ab, z1, (((2,), (1,)), ((0,), (0,))),
                                 preferred_element_type=jnp.float32)
        ccat = jnp.concatenate(
            [u, z1.reshape(t1 * _NP, _H), z2.reshape(t1 * _NP, _H)], axis=1)
        y = jnp.maximum(
            jnp.dot(ccat, wch_ref[...],
                    preferred_element_type=jnp.float32) + bch_ref[...], 0.0)

        h = _gated_tconv(y, t1, _H, wt2_ref, bt2_ref)

        ho_ref[i] = h
        h3 = h.reshape(t2, _NP, _H)
        hs = jnp.sum(h3, axis=0)
        h2s = jnp.sum(h3 * h3, axis=0)
        s1_ref[i] = jnp.sum(hs, axis=1, keepdims=True)
        s2_ref[i] = jnp.sum(h2s, axis=1, keepdims=True)


def _attn_body(tin, x_ref, affa_ref, affc_ref,
               win_ref, bin_ref, wout_ref, bout_ref, out_ref):
    aff_a = affa_ref[...].reshape(1, _NP, 1)
    aff_c = affc_ref[...].reshape(1, _NP, 1)

    bf = jnp.bfloat16
    tp = 56
    lane = jax.lax.broadcasted_iota(jnp.int32, (1, 1, _H), 2)
    masks = [((lane >= h * 16) & (lane < h * 16 + 16)).astype(jnp.float32)
             for h in range(4)]
    masks_b = [m.astype(bf) for m in masks]
    nid = jax.lax.broadcasted_iota(jnp.int32, (_NP, 1, 1), 0)

    for i in range(_NB):
        x3 = x_ref[i].astype(jnp.float32).reshape(tin, _NP, _H)
        xn = jnp.maximum(x3 * aff_a + aff_c, 0.0)
        xt = jnp.transpose(xn, (1, 0, 2)).astype(bf)
        winb = jnp.broadcast_to(win_ref[...], (_NP, _H, 3 * _H))
        qkv = jax.lax.dot_general(
            xt, winb, (((2,), (1,)), ((0,), (0,))),
            preferred_element_type=jnp.float32) + bin_ref[...]
        qn = (qkv[:, :, 0:_H] * 0.25).astype(bf)
        kn = qkv[:, :, _H:2 * _H].astype(bf)
        vn = qkv[:, :, 2 * _H:3 * _H].astype(bf)

        zpad = jnp.zeros((_NP, tp - tin, _H), bf)
        qm = jnp.concatenate(
            sum(([qn * m, zpad] for m in masks_b), []), axis=1)

        s = jax.lax.dot_general(qm, kn, (((2,), (2,)), ((0,), (0,))),
                                preferred_element_type=jnp.float32)
        e = jnp.exp(s)
        attn = (e * (1.0 / jnp.sum(e, axis=2, keepdims=True))).astype(bf)
        o4 = jax.lax.dot_general(attn, vn, (((2,), (1,)), ((0,), (0,))),
                                 preferred_element_type=jnp.float32)
        o = (o4[:, 0:tin] * masks[0]
             + o4[:, tp:tp + tin] * masks[1]
             + o4[:, 2 * tp:2 * tp + tin] * masks[2]
             + o4[:, 3 * tp:3 * tp + tin] * masks[3])

        om = jnp.where(nid < _N, o, 0.0)
        pv = (jnp.sum(jnp.sum(om, axis=0), axis=0, keepdims=True)
              * (1.0 / (_N * tin)))
        out_ref[i] = jnp.dot(pv, wout_ref[...]) + bout_ref[...]


def _heads_body(pooled_ref, ang_ref, wang_ref, bang_ref, wcls_ref, bcls_ref,
                wkey_ref, bkey_ref, wreg_ref, breg_ref,
                logits_ref, key_ref, angs_ref):
    xf = (pooled_ref[...] + jnp.dot(ang_ref[...], wang_ref[...])
          + bang_ref[...])
    logits_ref[...] = jnp.dot(xf, wcls_ref[...]) + bcls_ref[...]
    key_ref[...] = jax.nn.sigmoid(jnp.dot(xf, wkey_ref[...]) + bkey_ref[...])
    angs_ref[...] = jnp.dot(xf, wreg_ref[...]) + breg_ref[...]


def _full(shape):
    nd = len(shape)
    return pl.BlockSpec(shape, lambda b, _n=nd: (0,) * _n)


def _batched(shape):
    nd = len(shape)
    return pl.BlockSpec((_NB,) + shape[1:],
                        lambda b, _n=nd: (b,) + (0,) * (_n - 1))


def _tconv_w(p, w1, b1, w2, b2, w3, b3):
    def gate(w):
        return jnp.concatenate([w[:, :, 0, k].T for k in range(3)], axis=0)
    wcat = jnp.concatenate([gate(p[w1]), gate(p[w2]), gate(p[w3])], axis=1)
    bcat = jnp.concatenate([p[b1], p[b2], p[b3]])[None, :]
    return wcat, bcat


def kernel(X, angles, edge_index, params):
    f32 = jnp.float32
    bf = jnp.bfloat16

    ei_t = jnp.transpose(edge_index, (1, 0)).astype(jnp.int32)
    A = pl.pallas_call(
        _prep_body,
        out_shape=jax.ShapeDtypeStruct((_NP, _NP), f32),
    )(ei_t)

    layer_w = []
    for lp in params['layers']:
        wt1, bt1 = _tconv_w(lp['t1'], 'w1', 'b1', 'w2', 'b2', 'w3', 'b3')
        wt2, bt2 = _tconv_w(lp['t2'], 'w1', 'b1', 'w2', 'b2', 'w3', 'b3')
        w0, w1, w2 = lp['cheb_w']
        wch = jnp.concatenate([w0 - w2, w1, 2.0 * w2], axis=0)
        bch = lp['cheb_b'][None, :]
        g = jnp.pad(lp['bn_g'], (0, _NP - _N))[:, None]
        b = jnp.pad(lp['bn_b'], (0, _NP - _N))[:, None]
        layer_w.append((wt1, bt1, wch, bch, wt2, bt2, g, b))

    Xp = jnp.pad(X, ((0, 0), (0, 0), (0, _NP - _N), (0, 0)))
    Xr = Xp.reshape(_B, 64 * _NP, 2)

    tins = [64, 60, 56]
    h = Xr
    stats = None
    for li in range(3):
        tin = tins[li]
        t2 = tin - 4
        cin = 2 if li == 0 else _H
        wt1, bt1, wch, bch, wt2, bt2, g, b = layer_w[li]
        first = li == 0
        count = _B * tin * _H
        body = functools.partial(_layer_body, tin, cin, count, first)
        in_specs = [_batched((_B, tin * _NP, cin))]
        ins = [h]
        if not first:
            gp, bp = layer_w[li - 1][6], layer_w[li - 1][7]
            aff_a, aff_c = _bn_affine(count, stats[0], stats[1], gp, bp)
            in_specs += [_full((_NP, 1))] * 2
            ins += [aff_a, aff_c]
        in_specs += [_full((_NP, _NP)),
                     _full(wt1.shape), _full(bt1.shape),
                     _full(wch.shape), _full(bch.shape),
                     _full(wt2.shape), _full(bt2.shape)]
        ins += [A, wt1, bt1, wch, bch, wt2, bt2]
        h, s1, s2 = pl.pallas_call(
            body,
            grid=(_B // _NB,),
            in_specs=in_specs,
            out_specs=[_batched((_B, t2 * _NP, _H)),
                       _batched((_B, _NP, 1)), _batched((_B, _NP, 1))],
            out_shape=[jax.ShapeDtypeStruct((_B, t2 * _NP, _H), f32),
                       jax.ShapeDtypeStruct((_B, _NP, 1), f32),
                       jax.ShapeDtypeStruct((_B, _NP, 1), f32)],
        )(*ins)
        stats = (s1, s2)

    ap = params['attn']
    g3 = layer_w[2][6]
    b3 = layer_w[2][7]
    win = ap['in_w'].T.astype(bf)
    bin_ = ap['in_b'][None, :]
    wout = ap['out_w'].T
    bout = ap['out_b'][None, :]
    tin = 52
    aff_a, aff_c = _bn_affine(_B * tin * _H, stats[0], stats[1], g3, b3)
    body = functools.partial(_attn_body, tin)
    pooled = pl.pallas_call(
        body,
        grid=(_B // _NB,),
        in_specs=[_batched((_B, tin * _NP, _H)),
                  _full((_NP, 1)), _full((_NP, 1)),
                  _full(win.shape), _full(bin_.shape),
                  _full(wout.shape), _full(bout.shape)],
        out_specs=_batched((_B, 1, _H)),
        out_shape=jax.ShapeDtypeStruct((_B, 1, _H), f32),
    )(h, aff_a, aff_c, win, bin_, wout, bout)
    pooled = pooled.reshape(_B, _H)

    hp = params['heads']
    wang = hp['angle_w'].T
    bang = hp['angle_b'][None, :]
    wcls = hp['cls_w'].T
    bcls = hp['cls_b'][None, :]
    wkey = hp['key_w'].T
    bkey = hp['key_b'][None, :]
    wreg = hp['reg_w'].T
    breg = hp['reg_b'][None, :]
    logits, key_action, angle_scores = pl.pallas_call(
        _heads_body,
        out_shape=[jax.ShapeDtypeStruct((_B, 15), f32),
                   jax.ShapeDtypeStruct((_B, 1), f32),
                   jax.ShapeDtypeStruct((_B, 4), f32)],
    )(pooled, angles, wang, bang, wcls, bcls, wkey, bkey, wreg, breg)
    return (logits, key_action, angle_scores)


# speedup vs baseline: 3.5496x; 1.0003x over previous
import functools
import math

import jax
import jax.numpy as jnp
from jax.experimental import pallas as pl

_B = 128
_NB = 8
_N = 33
_NP = 40
_H = 64
_EPS = 1e-5


def _prep_body(ei_ref, a_ref):
    ei = ei_ref[...]
    row = ei[:, 0:1]
    col = ei[:, 1:2]
    n_iota = jax.lax.broadcasted_iota(jnp.int32, (70, _NP), 1)
    R = (row == n_iota).astype(jnp.float32)
    Cm = (col == n_iota).astype(jnp.float32)
    w = jnp.where(row == col, 0.0, 1.0)
    deg = jnp.sum(R * w, axis=0, keepdims=True)
    dis = jnp.where(deg > 0, jax.lax.rsqrt(jnp.maximum(deg, 1e-12)), 0.0)
    dr = jnp.sum(R * dis, axis=1, keepdims=True)
    dc = jnp.sum(Cm * dis, axis=1, keepdims=True)
    lapw = -dr * w * dc
    a_ref[...] = jax.lax.dot_general(
        Cm, lapw * R, (((0,), (0,)), ((), ())))


def _sigmoid(x):
    return 0.5 * jnp.tanh(0.5 * x) + 0.5


def _gated_tconv(x, tin, cin, w_ref, b_ref):
    t1 = tin - 2
    xcat = jnp.concatenate(
        [x[0:t1 * _NP], x[_NP:(t1 + 1) * _NP], x[2 * _NP:(t1 + 2) * _NP]],
        axis=1)
    gts = jnp.dot(xcat, w_ref[...],
                  preferred_element_type=jnp.float32) + b_ref[...]
    return jnp.maximum(
        gts[:, 0:_H] * _sigmoid(gts[:, _H:2 * _H]) + gts[:, 2 * _H:3 * _H],
        0.0)


def _stats_body(count, s1p_ref, s2p_ref, g_ref, b_ref, a_ref, c_ref):
    s1 = jnp.sum(s1p_ref[...], axis=0)
    s2 = jnp.sum(s2p_ref[...], axis=0)
    mean = s1 * (1.0 / count)
    var = s2 * (1.0 / count) - mean * mean
    inv = jax.lax.rsqrt(var + _EPS)
    a_ref[...] = g_ref[...] * inv
    c_ref[...] = b_ref[...] - mean * g_ref[...] * inv


def _bn_affine(count, s1, s2, g, b):
    return pl.pallas_call(
        functools.partial(_stats_body, count),
        out_shape=[jax.ShapeDtypeStruct((_NP, 1), jnp.float32)] * 2,
    )(s1, s2, g, b)


def _layer_body(tin, cin, count, first, *refs):
    if first:
        (x_ref, a_ref, wt1_ref, bt1_ref, wch_ref, bch_ref,
         wt2_ref, bt2_ref, ho_ref, s1_ref, s2_ref) = refs
    else:
        (x_ref, affa_ref, affc_ref, a_ref, wt1_ref, bt1_ref,
         wch_ref, bch_ref, wt2_ref, bt2_ref, ho_ref, s1_ref, s2_ref) = refs

    t1 = tin - 2
    t2 = t1 - 2
    A = a_ref[...]
    if not first:
        aff_a = affa_ref[...].reshape(1, _NP, 1)
        aff_c = affc_ref[...].reshape(1, _NP, 1)

    for i in range(_NB):
        x = x_ref[i]
        if not first:
            x3 = x.reshape(tin, _NP, cin)
            x = jnp.maximum(x3 * aff_a + aff_c, 0.0).reshape(tin * _NP, cin)

        u = _gated_tconv(x, tin, cin, wt1_ref, bt1_ref)

        u3 = u.reshape(t1, _NP, _H)
        Ab = jnp.broadcast_to(A, (t1, _NP, _NP))
        z1 = jax.lax.dot_general(Ab, u3, (((2,), (1,)), ((0,), (0,))),
                                 preferred_element_type=jnp.float32)
        z2 = jax.lax.dot_general(Ab, z1, (((2,), (1,)), ((0,), (0,))),
                                 preferred_element_type=jnp.float32)
        ccat = jnp.concatenate(
            [u, z1.reshape(t1 * _NP, _H), z2.reshape(t1 * _NP, _H)], axis=1)
        y = jnp.maximum(
            jnp.dot(ccat, wch_ref[...],
                    preferred_element_type=jnp.float32) + bch_ref[...], 0.0)

        h = _gated_tconv(y, t1, _H, wt2_ref, bt2_ref)

        ho_ref[i] = h
        h3 = h.reshape(t2, _NP, _H)
        hs = jnp.sum(h3, axis=0)
        h2s = jnp.sum(h3 * h3, axis=0)
        s1_ref[i] = jnp.sum(hs, axis=1, keepdims=True)
        s2_ref[i] = jnp.sum(h2s, axis=1, keepdims=True)


def _attn_body(tin, x_ref, affa_ref, affc_ref,
               win_ref, bin_ref, wout_ref, bout_ref, out_ref):
    aff_a = affa_ref[...].reshape(1, _NP, 1)
    aff_c = affc_ref[...].reshape(1, _NP, 1)

    bf = jnp.bfloat16
    tp = 56
    lane = jax.lax.broadcasted_iota(jnp.int32, (1, 1, _H), 2)
    masks = [((lane >= h * 16) & (lane < h * 16 + 16)).astype(jnp.float32)
             for h in range(4)]
    masks_b = [m.astype(bf) for m in masks]
    nid = jax.lax.broadcasted_iota(jnp.int32, (_NP, 1, 1), 0)

    ones_j = jnp.ones((_NP, _H, _H), bf)
    padbias = jnp.where(
        jax.lax.broadcasted_iota(jnp.int32, (1, 1, _H), 2) < tin,
        0.0, -1e30)
    kvpad = jnp.zeros((_NP, _H - tin, _H), bf)
    for i in range(_NB):
        x3 = x_ref[i].astype(jnp.float32).reshape(tin, _NP, _H)
        xn = jnp.maximum(x3 * aff_a + aff_c, 0.0).astype(bf)
        xt = jnp.transpose(xn, (1, 0, 2))
        winb = jnp.broadcast_to(win_ref[...], (_NP, _H, 3 * _H))
        qkv = jax.lax.dot_general(
            xt, winb, (((2,), (1,)), ((0,), (0,))),
            preferred_element_type=jnp.float32) + bin_ref[...]
        qn = (qkv[:, :, 0:_H] * 0.25).astype(bf)
        kn = jnp.concatenate([qkv[:, :, _H:2 * _H].astype(bf), kvpad], axis=1)
        vn = jnp.concatenate([qkv[:, :, 2 * _H:3 * _H].astype(bf), kvpad],
                             axis=1)

        zpad = jnp.zeros((_NP, tp - tin, _H), bf)
        qm = jnp.concatenate(
            sum(([qn * m, zpad] for m in masks_b), []), axis=1)

        s = jax.lax.dot_general(qm, kn, (((2,), (2,)), ((0,), (0,))),
                                preferred_element_type=jnp.float32) + padbias
        e = jnp.exp(s).astype(bf)
        ssum = jax.lax.dot_general(e, ones_j, (((2,), (1,)), ((0,), (0,))),
                                   preferred_element_type=jnp.float32)
        o4 = jax.lax.dot_general(e, vn, (((2,), (1,)), ((0,), (0,))),
                                 preferred_element_type=jnp.float32)
        o4n = o4 / ssum
        o = (o4n[:, 0:tin] * masks[0]
             + o4n[:, tp:tp + tin] * masks[1]
             + o4n[:, 2 * tp:2 * tp + tin] * masks[2]
             + o4n[:, 3 * tp:3 * tp + tin] * masks[3])

        om = jnp.where(nid < _N, o, 0.0)
        pv = (jnp.sum(jnp.sum(om, axis=0), axis=0, keepdims=True)
              * (1.0 / (_N * tin)))
        out_ref[i] = jnp.dot(pv, wout_ref[...]) + bout_ref[...]


def _heads_body(pooled_ref, ang_ref, wang_ref, bang_ref, wcls_ref, bcls_ref,
                wkey_ref, bkey_ref, wreg_ref, breg_ref,
                logits_ref, key_ref, angs_ref):
    xf = (pooled_ref[...] + jnp.dot(ang_ref[...], wang_ref[...])
          + bang_ref[...])
    logits_ref[...] = jnp.dot(xf, wcls_ref[...]) + bcls_ref[...]
    key_ref[...] = jax.nn.sigmoid(jnp.dot(xf, wkey_ref[...]) + bkey_ref[...])
    angs_ref[...] = jnp.dot(xf, wreg_ref[...]) + breg_ref[...]


def _full(shape):
    nd = len(shape)
    return pl.BlockSpec(shape, lambda b, _n=nd: (0,) * _n)


def _batched(shape):
    nd = len(shape)
    return pl.BlockSpec((_NB,) + shape[1:],
                        lambda b, _n=nd: (b,) + (0,) * (_n - 1))


def _tconv_w(p, w1, b1, w2, b2, w3, b3):
    def gate(w):
        return jnp.concatenate([w[:, :, 0, k].T for k in range(3)], axis=0)
    wcat = jnp.concatenate([gate(p[w1]), gate(p[w2]), gate(p[w3])], axis=1)
    bcat = jnp.concatenate([p[b1], p[b2], p[b3]])[None, :]
    return wcat, bcat


def kernel(X, angles, edge_index, params):
    f32 = jnp.float32
    bf = jnp.bfloat16

    ei_t = jnp.transpose(edge_index, (1, 0)).astype(jnp.int32)
    A = pl.pallas_call(
        _prep_body,
        out_shape=jax.ShapeDtypeStruct((_NP, _NP), f32),
    )(ei_t)

    layer_w = []
    for lp in params['layers']:
        wt1, bt1 = _tconv_w(lp['t1'], 'w1', 'b1', 'w2', 'b2', 'w3', 'b3')
        wt2, bt2 = _tconv_w(lp['t2'], 'w1', 'b1', 'w2', 'b2', 'w3', 'b3')
        w0, w1, w2 = lp['cheb_w']
        wch = jnp.concatenate([w0 - w2, w1, 2.0 * w2], axis=0)
        bch = lp['cheb_b'][None, :]
        g = jnp.pad(lp['bn_g'], (0, _NP - _N))[:, None]
        b = jnp.pad(lp['bn_b'], (0, _NP - _N))[:, None]
        layer_w.append((wt1, bt1, wch, bch, wt2, bt2, g, b))

    Xp = jnp.pad(X, ((0, 0), (0, 0), (0, _NP - _N), (0, 0)))
    Xr = Xp.reshape(_B, 64 * _NP, 2)

    tins = [64, 60, 56]
    h = Xr
    stats = None
    for li in range(3):
        tin = tins[li]
        t2 = tin - 4
        cin = 2 if li == 0 else _H
        wt1, bt1, wch, bch, wt2, bt2, g, b = layer_w[li]
        first = li == 0
        count = _B * tin * _H
        body = functools.partial(_layer_body, tin, cin, count, first)
        in_specs = [_batched((_B, tin * _NP, cin))]
        ins = [h]
        if not first:
            gp, bp = layer_w[li - 1][6], layer_w[li - 1][7]
            aff_a, aff_c = _bn_affine(count, stats[0], stats[1], gp, bp)
            in_specs += [_full((_NP, 1))] * 2
            ins += [aff_a, aff_c]
        in_specs += [_full((_NP, _NP)),
                     _full(wt1.shape), _full(bt1.shape),
                     _full(wch.shape), _full(bch.shape),
                     _full(wt2.shape), _full(bt2.shape)]
        ins += [A, wt1, bt1, wch, bch, wt2, bt2]
        h, s1, s2 = pl.pallas_call(
            body,
            grid=(_B // _NB,),
            in_specs=in_specs,
            out_specs=[_batched((_B, t2 * _NP, _H)),
                       _batched((_B, _NP, 1)), _batched((_B, _NP, 1))],
            out_shape=[jax.ShapeDtypeStruct((_B, t2 * _NP, _H), f32),
                       jax.ShapeDtypeStruct((_B, _NP, 1), f32),
                       jax.ShapeDtypeStruct((_B, _NP, 1), f32)],
        )(*ins)
        stats = (s1, s2)

    ap = params['attn']
    g3 = layer_w[2][6]
    b3 = layer_w[2][7]
    win = ap['in_w'].T.astype(bf)
    bin_ = ap['in_b'][None, :]
    wout = ap['out_w'].T
    bout = ap['out_b'][None, :]
    tin = 52
    aff_a, aff_c = _bn_affine(_B * tin * _H, stats[0], stats[1], g3, b3)
    body = functools.partial(_attn_body, tin)
    pooled = pl.pallas_call(
        body,
        grid=(_B // _NB,),
        in_specs=[_batched((_B, tin * _NP, _H)),
                  _full((_NP, 1)), _full((_NP, 1)),
                  _full(win.shape), _full(bin_.shape),
                  _full(wout.shape), _full(bout.shape)],
        out_specs=_batched((_B, 1, _H)),
        out_shape=jax.ShapeDtypeStruct((_B, 1, _H), f32),
    )(h, aff_a, aff_c, win, bin_, wout, bout)
    pooled = pooled.reshape(_B, _H)

    hp = params['heads']
    wang = hp['angle_w'].T
    bang = hp['angle_b'][None, :]
    wcls = hp['cls_w'].T
    bcls = hp['cls_b'][None, :]
    wkey = hp['key_w'].T
    bkey = hp['key_b'][None, :]
    wreg = hp['reg_w'].T
    breg = hp['reg_b'][None, :]
    logits, key_action, angle_scores = pl.pallas_call(
        _heads_body,
        out_shape=[jax.ShapeDtypeStruct((_B, 15), f32),
                   jax.ShapeDtypeStruct((_B, 1), f32),
                   jax.ShapeDtypeStruct((_B, 4), f32)],
    )(pooled, angles, wang, bang, wcls, bcls, wkey, bkey, wreg, breg)
    return (logits, key_action, angle_scores)
